# Initial kernel scaffold; baseline (speedup 1.0000x reference)
import jax
import jax.numpy as jnp
from jax.experimental import pallas as pl


def kernel(raw, edge_index, W_mlp, b_mlp, W_out, b_out, W_root, ln_g, ln_b, W_post, b_post, W_lin1, b_lin1):
    raise NotImplementedError("write your pallas kernel here")


# speedup vs baseline: 15.9281x; 15.9281x over previous
import functools

import jax
import jax.numpy as jnp
from jax import lax
from jax.experimental import pallas as pl
from jax.experimental.pallas import tpu as pltpu
from jax.experimental.pallas import tpu_sc as plsc

N_NODES = 10000
N_EDGES = 320000
D_FEAT = 128
HIDDEN = 128
N_CLASSES = 64
LN_EPS = 1e-5

NCORE = 2
NSUB = 16
NW = NCORE * NSUB
EPT = N_EDGES // NW
K = 80
NCHUNK = EPT // K
AGG_ROWS = 10240
RPT = AGG_ROWS // NSUB
CPT = AGG_ROWS // NSUB

BROW = 1000

_INV_SQRT2 = 0.7071067811865476


def _gelu_exact(x):
    return 0.5 * x * (1.0 + lax.erf(x * _INV_SQRT2))


def _mlp_body(raw_ref, wm_ref, bm_ref, out_ref):
    xb = lax.dot_general(raw_ref[...], wm_ref[...],
                         (((1,), (1,)), ((), ())),
                         preferred_element_type=jnp.float32)
    out_ref[...] = _gelu_exact(xb + bm_ref[...])


def _deg_body(row_hbm, col_hbm, deg_hbm, row_v, col_v, deg_v, dtmp_v, dres_v,
              stage_sh):
    cid = lax.axis_index("c")
    sid = lax.axis_index("s")
    wid = cid * jnp.int32(NSUB) + sid

    pltpu.sync_copy(row_hbm.at[wid], row_v)
    pltpu.sync_copy(col_hbm.at[wid], col_v)

    zero = jnp.zeros((16,), jnp.float32)
    ones = jnp.ones((16,), jnp.float32)

    @pl.loop(jnp.int32(0), jnp.int32(AGG_ROWS // 16))
    def _zd(i):
        deg_v[pl.ds(i * jnp.int32(16), 16)] = zero

    @pl.loop(jnp.int32(0), jnp.int32(NCHUNK))
    def _hist(j):
        for k2 in range(K // 16):
            sl = pl.ds(jnp.int32(k2 * 16), 16)
            r = row_v[j, sl]
            cc = col_v[j, sl]
            plsc.addupdate_scatter(deg_v, [cc], lax.select(r != cc, ones, zero))

    pltpu.sync_copy(deg_v, stage_sh.at[sid])
    plsc.subcore_barrier()

    cbase = sid * jnp.int32(CPT)
    for t in range(NSUB):
        pltpu.sync_copy(stage_sh.at[jnp.int32(t), pl.ds(cbase, CPT)],
                        dtmp_v.at[jnp.int32(t)])

    @pl.loop(jnp.int32(0), jnp.int32(CPT // 16))
    def _dsum(k2):
        base = k2 * jnp.int32(16)
        acc = dtmp_v[jnp.int32(0), pl.ds(base, 16)]
        for t in range(1, NSUB):
            acc = acc + dtmp_v[jnp.int32(t), pl.ds(base, 16)]
        dres_v[pl.ds(base, 16)] = acc

    pltpu.sync_copy(dres_v,
                    deg_hbm.at[pl.ds(cid * jnp.int32(AGG_ROWS) + cbase, CPT)])


def _agg_body(x_hbm, row_hbm, col_hbm, out_hbm,
              row_v, col_v, buf_a, agg_sh, sem_a):
    cid = lax.axis_index("c")
    sid = lax.axis_index("s")
    wid = cid * jnp.int32(NSUB) + sid

    pltpu.sync_copy(row_hbm.at[wid], row_v)
    pltpu.sync_copy(col_hbm.at[wid], col_v)

    zero = jnp.zeros((16,), jnp.float32)

    @pl.loop(jnp.int32(0), jnp.int32(K))
    def _zb(i):
        for k2 in range(D_FEAT // 16):
            buf_a[i, pl.ds(jnp.int32(k2 * 16), 16)] = zero

    for b in range(RPT // K):
        pltpu.sync_copy(
            buf_a, agg_sh.at[pl.ds(sid * jnp.int32(RPT) + jnp.int32(b * K), K)])

    @pl.loop(jnp.int32(0), jnp.int32(NCHUNK))
    def _cm(j):
        for k2 in range(K // 16):
            sl = pl.ds(jnp.int32(k2 * 16), 16)
            r = row_v[j, sl]
            cc = col_v[j, sl]
            col_v[j, sl] = jnp.where(r == cc, jnp.int32(N_NODES), cc)

    plsc.subcore_barrier()

    @pl.loop(jnp.int32(0), jnp.int32(NCHUNK))
    def _mainb(j):
        pltpu.async_copy(x_hbm.at[row_v.at[j]], buf_a, sem_a).wait()
        pltpu.sync_copy(buf_a, agg_sh.at[col_v.at[j]], add=True)

    plsc.subcore_barrier()

    base = cid * jnp.int32(AGG_ROWS) + sid * jnp.int32(RPT)
    pltpu.sync_copy(agg_sh.at[pl.ds(sid * jnp.int32(RPT), RPT)],
                    out_hbm.at[pl.ds(base, RPT)])


def _tail_body(raw_ref, xa_ref, s0_ref, s1_ref, d0_ref, d1_ref,
               wo_ref, bo_ref, wr_ref,
               g_ref, b_ref, wp_ref, bp_ref, wl_ref, bl_ref,
               out_ref, emb_ref):
    xb = xa_ref[...]
    s = s0_ref[...] + s1_ref[...]
    deg = d0_ref[...] + d1_ref[...]
    deg_inv = 1.0 / (deg + 1.0)
    agg = deg_inv * (s + xb)
    ne = (lax.dot_general(agg, wo_ref[...], (((1,), (1,)), ((), ())),
                          preferred_element_type=jnp.float32)
          + bo_ref[...]
          + lax.dot_general(xb, wr_ref[...], (((1,), (1,)), ((), ())),
                            preferred_element_type=jnp.float32))
    emb_ref[...] = ne
    h = _gelu_exact(ne)
    hc = jnp.concatenate([raw_ref[...], h], axis=1)
    mu = jnp.mean(hc, axis=-1, keepdims=True)
    var = jnp.mean((hc - mu) ** 2, axis=-1, keepdims=True)
    ln = (hc - mu) / jnp.sqrt(var + LN_EPS) * g_ref[...] + b_ref[...]
    h2 = _gelu_exact(
        lax.dot_general(ln, wp_ref[...], (((1,), (1,)), ((), ())),
                        preferred_element_type=jnp.float32) + bp_ref[...])
    out_ref[...] = jax.nn.sigmoid(
        lax.dot_general(h2, wl_ref[...], (((1,), (1,)), ((), ())),
                        preferred_element_type=jnp.float32) + bl_ref[...])


_full = lambda shape: pl.BlockSpec(shape, lambda i: (i * 0,) * len(shape))
_rows = lambda w: pl.BlockSpec((BROW, w), lambda i: (i, i * 0))

_mlp_call = pl.pallas_call(
    _mlp_body,
    grid=(N_NODES // BROW,),
    in_specs=[_rows(D_FEAT), _full((HIDDEN, D_FEAT)), _full((1, HIDDEN))],
    out_specs=_rows(D_FEAT),
    out_shape=jax.ShapeDtypeStruct((N_NODES, D_FEAT), jnp.float32),
)


@functools.cache
def _get_sc_calls():
    mesh = plsc.VectorSubcoreMesh(core_axis_name="c", subcore_axis_name="s",
                                  num_cores=NCORE)
    deg_call = functools.partial(
        pl.kernel,
        mesh=mesh,
        compiler_params=pltpu.CompilerParams(needs_layout_passes=False),
        out_type=jax.ShapeDtypeStruct((NCORE * AGG_ROWS,), jnp.float32),
        scratch_types=[
            pltpu.VMEM((NCHUNK, K), jnp.int32),
            pltpu.VMEM((NCHUNK, K), jnp.int32),
            pltpu.VMEM((AGG_ROWS,), jnp.float32),
            pltpu.VMEM((NSUB, CPT), jnp.float32),
            pltpu.VMEM((CPT,), jnp.float32),
            pltpu.VMEM_SHARED((NSUB, AGG_ROWS), jnp.float32),
        ],
    )(_deg_body)
    agg_call = functools.partial(
        pl.kernel,
        mesh=mesh,
        compiler_params=pltpu.CompilerParams(needs_layout_passes=False),
        out_type=jax.ShapeDtypeStruct((NCORE * AGG_ROWS, D_FEAT), jnp.float32),
        scratch_types=[
            pltpu.VMEM((NCHUNK, K), jnp.int32),
            pltpu.VMEM((NCHUNK, K), jnp.int32),
            pltpu.VMEM((K, D_FEAT), jnp.float32),
            pltpu.VMEM_SHARED((AGG_ROWS, D_FEAT), jnp.float32),
            pltpu.SemaphoreType.DMA,
        ],
    )(_agg_body)
    return deg_call, agg_call


_tail_call = pl.pallas_call(
    _tail_body,
    grid=(N_NODES // BROW,),
    in_specs=[
        _rows(D_FEAT),
        _rows(D_FEAT),
        _rows(D_FEAT),
        _rows(D_FEAT),
        _rows(1),
        _rows(1),
        _full((HIDDEN, HIDDEN)),
        _full((1, HIDDEN)),
        _full((HIDDEN, HIDDEN)),
        _full((1, D_FEAT + HIDDEN)),
        _full((1, D_FEAT + HIDDEN)),
        _full((HIDDEN, D_FEAT + HIDDEN)),
        _full((1, HIDDEN)),
        _full((N_CLASSES, HIDDEN)),
        _full((1, N_CLASSES)),
    ],
    out_specs=[_rows(N_CLASSES), _rows(HIDDEN)],
    out_shape=[
        jax.ShapeDtypeStruct((N_NODES, N_CLASSES), jnp.float32),
        jax.ShapeDtypeStruct((N_NODES, HIDDEN), jnp.float32),
    ],
)


def kernel(raw, edge_index, W_mlp, b_mlp, W_out, b_out, W_root, ln_g, ln_b,
           W_post, b_post, W_lin1, b_lin1):
    ei = edge_index.astype(jnp.int32)
    row3 = ei[0].reshape(NW, NCHUNK, K)
    col3 = ei[1].reshape(NW, NCHUNK, K)

    x = _mlp_call(raw, W_mlp, b_mlp.reshape(1, HIDDEN))

    deg_call, agg_call = _get_sc_calls()
    deg_flat = deg_call(row3, col3)
    s_flat = agg_call(x, row3, col3)
    s0 = s_flat[:N_NODES]
    s1 = s_flat[AGG_ROWS:AGG_ROWS + N_NODES]
    degr = deg_flat.reshape(NCORE, AGG_ROWS)
    d0 = degr[0, :N_NODES].reshape(N_NODES, 1)
    d1 = degr[1, :N_NODES].reshape(N_NODES, 1)

    out, node_emb = _tail_call(
        raw, x, s0, s1, d0, d1,
        W_out, b_out.reshape(1, HIDDEN), W_root,
        ln_g.reshape(1, D_FEAT + HIDDEN), ln_b.reshape(1, D_FEAT + HIDDEN),
        W_post, b_post.reshape(1, HIDDEN),
        W_lin1, b_lin1.reshape(1, N_CLASSES))
    return out, node_emb


# speedup vs baseline: 19.0867x; 1.1983x over previous
import functools

import jax
import jax.numpy as jnp
from jax import lax
from jax.experimental import pallas as pl
from jax.experimental.pallas import tpu as pltpu
from jax.experimental.pallas import tpu_sc as plsc

N_NODES = 10000
N_EDGES = 320000
D_FEAT = 128
HIDDEN = 128
N_CLASSES = 64
LN_EPS = 1e-5

NCORE = 2
NSUB = 16
NW = NCORE * NSUB
EPT = N_EDGES // NW
K = 80
NCHUNK = EPT // K
AGG_ROWS = 10240
RPT = AGG_ROWS // NSUB
CPT = AGG_ROWS // NSUB

BROW = 1000

_INV_SQRT2 = 0.7071067811865476


def _gelu_exact(x):
    return 0.5 * x * (1.0 + lax.erf(x * _INV_SQRT2))


def _mlp_body(raw_ref, wm_ref, bm_ref, out_ref):
    xb = lax.dot_general(raw_ref[...], wm_ref[...],
                         (((1,), (1,)), ((), ())),
                         preferred_element_type=jnp.float32)
    out_ref[...] = _gelu_exact(xb + bm_ref[...])


def _deg_body(row_hbm, col_hbm, deg_hbm, row_v, col_v, deg_v, dtmp_v, dres_v,
              stage_sh):
    cid = lax.axis_index("c")
    sid = lax.axis_index("s")
    wid = cid * jnp.int32(NSUB) + sid

    pltpu.sync_copy(row_hbm.at[wid], row_v)
    pltpu.sync_copy(col_hbm.at[wid], col_v)

    zero = jnp.zeros((16,), jnp.float32)
    ones = jnp.ones((16,), jnp.float32)

    @pl.loop(jnp.int32(0), jnp.int32(AGG_ROWS // 16))
    def _zd(i):
        deg_v[pl.ds(i * jnp.int32(16), 16)] = zero

    @pl.loop(jnp.int32(0), jnp.int32(NCHUNK))
    def _hist(j):
        for k2 in range(K // 16):
            sl = pl.ds(jnp.int32(k2 * 16), 16)
            r = row_v[j, sl]
            cc = col_v[j, sl]
            plsc.addupdate_scatter(deg_v, [cc], lax.select(r != cc, ones, zero))

    pltpu.sync_copy(deg_v, stage_sh.at[sid])
    plsc.subcore_barrier()

    cbase = sid * jnp.int32(CPT)
    for t in range(NSUB):
        pltpu.sync_copy(stage_sh.at[jnp.int32(t), pl.ds(cbase, CPT)],
                        dtmp_v.at[jnp.int32(t)])

    @pl.loop(jnp.int32(0), jnp.int32(CPT // 16))
    def _dsum(k2):
        base = k2 * jnp.int32(16)
        acc = dtmp_v[jnp.int32(0), pl.ds(base, 16)]
        for t in range(1, NSUB):
            acc = acc + dtmp_v[jnp.int32(t), pl.ds(base, 16)]
        dres_v[pl.ds(base, 16)] = acc

    pltpu.sync_copy(dres_v,
                    deg_hbm.at[pl.ds(cid * jnp.int32(AGG_ROWS) + cbase, CPT)])


def _agg_body(x_hbm, row_hbm, col_hbm, out_hbm,
              row_v, col_v, buf_a, buf_b, agg_sh, sem_a, sem_b):
    cid = lax.axis_index("c")
    sid = lax.axis_index("s")
    wid = cid * jnp.int32(NSUB) + sid

    pltpu.sync_copy(row_hbm.at[wid], row_v)
    pltpu.sync_copy(col_hbm.at[wid], col_v)

    zero = jnp.zeros((16,), jnp.float32)

    @pl.loop(jnp.int32(0), jnp.int32(K))
    def _zb(i):
        for k2 in range(D_FEAT // 16):
            buf_a[i, pl.ds(jnp.int32(k2 * 16), 16)] = zero

    for b in range(RPT // K):
        pltpu.sync_copy(
            buf_a, agg_sh.at[pl.ds(sid * jnp.int32(RPT) + jnp.int32(b * K), K)])

    @pl.loop(jnp.int32(0), jnp.int32(NCHUNK))
    def _cm(j):
        for k2 in range(K // 16):
            r = row_v[pl.ds(j * jnp.int32(K) + jnp.int32(k2 * 16), 16)]
            sl = pl.ds(jnp.int32(k2 * 16), 16)
            cc = col_v[j, sl]
            col_v[j, sl] = jnp.where(r == cc, jnp.int32(N_NODES), cc)

    plsc.subcore_barrier()

    def _gather(j, buf, sem):
        return pltpu.async_copy(x_hbm.at[row_v.at[pl.ds(j * jnp.int32(K), K)]],
                                buf, sem)

    def _scatter(j, buf):
        pltpu.sync_copy(buf, agg_sh.at[col_v.at[j]], add=True)

    _gather(jnp.int32(0), buf_a, sem_a).wait()

    @pl.loop(jnp.int32(0), jnp.int32(NCHUNK - 1), step=jnp.int32(2))
    def _mainb(j):
        g_b = _gather(j + jnp.int32(1), buf_b, sem_b)
        _scatter(j, buf_a)
        g_b.wait()
        g_a = _gather(j + jnp.int32(2), buf_a, sem_a)
        _scatter(j + jnp.int32(1), buf_b)
        g_a.wait()

    _scatter(jnp.int32(NCHUNK - 1), buf_a)

    plsc.subcore_barrier()

    base = cid * jnp.int32(AGG_ROWS) + sid * jnp.int32(RPT)
    pltpu.sync_copy(agg_sh.at[pl.ds(sid * jnp.int32(RPT), RPT)],
                    out_hbm.at[pl.ds(base, RPT)])


def _tail_body(raw_ref, xa_ref, s0_ref, s1_ref, d0_ref, d1_ref,
               wo_ref, bo_ref, wr_ref,
               g_ref, b_ref, wp_ref, bp_ref, wl_ref, bl_ref,
               out_ref, emb_ref):
    xb = xa_ref[...]
    s = s0_ref[...] + s1_ref[...]
    deg = d0_ref[...] + d1_ref[...]
    deg_inv = 1.0 / (deg + 1.0)
    agg = deg_inv * (s + xb)
    ne = (lax.dot_general(agg, wo_ref[...], (((1,), (1,)), ((), ())),
                          preferred_element_type=jnp.float32)
          + bo_ref[...]
          + lax.dot_general(xb, wr_ref[...], (((1,), (1,)), ((), ())),
                            preferred_element_type=jnp.float32))
    emb_ref[...] = ne
    h = _gelu_exact(ne)
    hc = jnp.concatenate([raw_ref[...], h], axis=1)
    mu = jnp.mean(hc, axis=-1, keepdims=True)
    var = jnp.mean((hc - mu) ** 2, axis=-1, keepdims=True)
    ln = (hc - mu) / jnp.sqrt(var + LN_EPS) * g_ref[...] + b_ref[...]
    h2 = _gelu_exact(
        lax.dot_general(ln, wp_ref[...], (((1,), (1,)), ((), ())),
                        preferred_element_type=jnp.float32) + bp_ref[...])
    out_ref[...] = jax.nn.sigmoid(
        lax.dot_general(h2, wl_ref[...], (((1,), (1,)), ((), ())),
                        preferred_element_type=jnp.float32) + bl_ref[...])


_full = lambda shape: pl.BlockSpec(shape, lambda i: (i * 0,) * len(shape))
_rows = lambda w: pl.BlockSpec((BROW, w), lambda i: (i, i * 0))

_mlp_call = pl.pallas_call(
    _mlp_body,
    grid=(N_NODES // BROW,),
    in_specs=[_rows(D_FEAT), _full((HIDDEN, D_FEAT)), _full((1, HIDDEN))],
    out_specs=_rows(D_FEAT),
    out_shape=jax.ShapeDtypeStruct((N_NODES, D_FEAT), jnp.float32),
)


@functools.cache
def _get_sc_calls():
    mesh = plsc.VectorSubcoreMesh(core_axis_name="c", subcore_axis_name="s",
                                  num_cores=NCORE)
    deg_call = functools.partial(
        pl.kernel,
        mesh=mesh,
        compiler_params=pltpu.CompilerParams(needs_layout_passes=False),
        out_type=jax.ShapeDtypeStruct((NCORE * AGG_ROWS,), jnp.float32),
        scratch_types=[
            pltpu.VMEM((NCHUNK, K), jnp.int32),
            pltpu.VMEM((NCHUNK, K), jnp.int32),
            pltpu.VMEM((AGG_ROWS,), jnp.float32),
            pltpu.VMEM((NSUB, CPT), jnp.float32),
            pltpu.VMEM((CPT,), jnp.float32),
            pltpu.VMEM_SHARED((NSUB, AGG_ROWS), jnp.float32),
        ],
    )(_deg_body)
    agg_call = functools.partial(
        pl.kernel,
        mesh=mesh,
        compiler_params=pltpu.CompilerParams(needs_layout_passes=False),
        out_type=jax.ShapeDtypeStruct((NCORE * AGG_ROWS, D_FEAT), jnp.float32),
        scratch_types=[
            pltpu.VMEM((EPT,), jnp.int32),
            pltpu.VMEM((NCHUNK, K), jnp.int32),
            pltpu.VMEM((K, D_FEAT), jnp.float32),
            pltpu.VMEM((K, D_FEAT), jnp.float32),
            pltpu.VMEM_SHARED((AGG_ROWS, D_FEAT), jnp.float32),
            pltpu.SemaphoreType.DMA,
            pltpu.SemaphoreType.DMA,
        ],
    )(_agg_body)
    return deg_call, agg_call


_tail_call = pl.pallas_call(
    _tail_body,
    grid=(N_NODES // BROW,),
    in_specs=[
        _rows(D_FEAT),
        _rows(D_FEAT),
        _rows(D_FEAT),
        _rows(D_FEAT),
        _rows(1),
        _rows(1),
        _full((HIDDEN, HIDDEN)),
        _full((1, HIDDEN)),
        _full((HIDDEN, HIDDEN)),
        _full((1, D_FEAT + HIDDEN)),
        _full((1, D_FEAT + HIDDEN)),
        _full((HIDDEN, D_FEAT + HIDDEN)),
        _full((1, HIDDEN)),
        _full((N_CLASSES, HIDDEN)),
        _full((1, N_CLASSES)),
    ],
    out_specs=[_rows(N_CLASSES), _rows(HIDDEN)],
    out_shape=[
        jax.ShapeDtypeStruct((N_NODES, N_CLASSES), jnp.float32),
        jax.ShapeDtypeStruct((N_NODES, HIDDEN), jnp.float32),
    ],
)


def kernel(raw, edge_index, W_mlp, b_mlp, W_out, b_out, W_root, ln_g, ln_b,
           W_post, b_post, W_lin1, b_lin1):
    ei = edge_index.astype(jnp.int32)
    row2 = ei[0].reshape(NW, EPT)
    row3 = ei[0].reshape(NW, NCHUNK, K)
    col3 = ei[1].reshape(NW, NCHUNK, K)

    x = _mlp_call(raw, W_mlp, b_mlp.reshape(1, HIDDEN))

    deg_call, agg_call = _get_sc_calls()
    deg_flat = deg_call(row3, col3)
    s_flat = agg_call(x, row2, col3)
    s0 = s_flat[:N_NODES]
    s1 = s_flat[AGG_ROWS:AGG_ROWS + N_NODES]
    degr = deg_flat.reshape(NCORE, AGG_ROWS)
    d0 = degr[0, :N_NODES].reshape(N_NODES, 1)
    d1 = degr[1, :N_NODES].reshape(N_NODES, 1)

    out, node_emb = _tail_call(
        raw, x, s0, s1, d0, d1,
        W_out, b_out.reshape(1, HIDDEN), W_root,
        ln_g.reshape(1, D_FEAT + HIDDEN), ln_b.reshape(1, D_FEAT + HIDDEN),
        W_post, b_post.reshape(1, HIDDEN),
        W_lin1, b_lin1.reshape(1, N_CLASSES))
    return out, node_emb


# speedup vs baseline: 19.6964x; 1.0319x over previous
import functools

import jax
import jax.numpy as jnp
from jax import lax
from jax.experimental import pallas as pl
from jax.experimental.pallas import tpu as pltpu
from jax.experimental.pallas import tpu_sc as plsc

N_NODES = 10000
N_EDGES = 320000
D_FEAT = 128
HIDDEN = 128
N_CLASSES = 64
LN_EPS = 1e-5

NCORE = 2
NSUB = 16
NW = NCORE * NSUB
EPT = N_EDGES // NW
K = 80
NCHUNK = EPT // K
AGG_ROWS = 10240
RPT = AGG_ROWS // NSUB
CPT = AGG_ROWS // NSUB

BROW = 1000

_INV_SQRT2 = 0.7071067811865476


def _gelu_exact(x):
    return 0.5 * x * (1.0 + lax.erf(x * _INV_SQRT2))


def _mlp_body(raw_ref, wm_ref, bm_ref, out_ref):
    xb = lax.dot_general(raw_ref[...], wm_ref[...],
                         (((1,), (1,)), ((), ())),
                         preferred_element_type=jnp.float32)
    out_ref[...] = _gelu_exact(xb + bm_ref[...])


def _deg_body(row_hbm, col_hbm, deg_hbm, row_v, col_v, deg_v, dtmp_v, dres_v,
              stage_sh):
    cid = lax.axis_index("c")
    sid = lax.axis_index("s")
    wid = cid * jnp.int32(NSUB) + sid

    pltpu.sync_copy(row_hbm.at[wid], row_v)
    pltpu.sync_copy(col_hbm.at[wid], col_v)

    zero = jnp.zeros((16,), jnp.float32)
    ones = jnp.ones((16,), jnp.float32)

    @pl.loop(jnp.int32(0), jnp.int32(AGG_ROWS // 16))
    def _zd(i):
        deg_v[pl.ds(i * jnp.int32(16), 16)] = zero

    @pl.loop(jnp.int32(0), jnp.int32(NCHUNK))
    def _hist(j):
        for k2 in range(K // 16):
            sl = pl.ds(jnp.int32(k2 * 16), 16)
            r = row_v[j, sl]
            cc = col_v[j, sl]
            plsc.addupdate_scatter(deg_v, [cc], lax.select(r != cc, ones, zero))

    pltpu.sync_copy(deg_v, stage_sh.at[sid])
    plsc.subcore_barrier()

    cbase = sid * jnp.int32(CPT)
    for t in range(NSUB):
        pltpu.sync_copy(stage_sh.at[jnp.int32(t), pl.ds(cbase, CPT)],
                        dtmp_v.at[jnp.int32(t)])

    @pl.loop(jnp.int32(0), jnp.int32(CPT // 16))
    def _dsum(k2):
        base = k2 * jnp.int32(16)
        acc = dtmp_v[jnp.int32(0), pl.ds(base, 16)]
        for t in range(1, NSUB):
            acc = acc + dtmp_v[jnp.int32(t), pl.ds(base, 16)]
        dres_v[pl.ds(base, 16)] = acc

    pltpu.sync_copy(dres_v,
                    deg_hbm.at[pl.ds(cid * jnp.int32(AGG_ROWS) + cbase, CPT)])


def _agg_body(x_hbm, row_hbm, col_hbm, out_hbm,
              row_v, col_v, buf_a, buf_b, agg_sh, sem_a, sem_b):
    cid = lax.axis_index("c")
    sid = lax.axis_index("s")
    wid = cid * jnp.int32(NSUB) + sid

    pltpu.sync_copy(row_hbm.at[wid], row_v)
    pltpu.sync_copy(col_hbm.at[wid], col_v)

    zero = jnp.zeros((16,), jnp.float32)

    @pl.loop(jnp.int32(0), jnp.int32(K))
    def _zb(i):
        for k2 in range(D_FEAT // 16):
            buf_a[i, pl.ds(jnp.int32(k2 * 16), 16)] = zero

    for b in range(RPT // K):
        pltpu.sync_copy(
            buf_a, agg_sh.at[pl.ds(sid * jnp.int32(RPT) + jnp.int32(b * K), K)])

    @pl.loop(jnp.int32(0), jnp.int32(NCHUNK))
    def _cm(j):
        for k2 in range(K // 16):
            r = row_v[pl.ds(j * jnp.int32(K) + jnp.int32(k2 * 16), 16)]
            sl = pl.ds(jnp.int32(k2 * 16), 16)
            cc = col_v[j, sl]
            col_v[j, sl] = jnp.where(r == cc, jnp.int32(N_NODES), cc)

    plsc.subcore_barrier()

    def _gather(j, buf, sem):
        return pltpu.async_copy(x_hbm.at[row_v.at[pl.ds(j * jnp.int32(K), K)]],
                                buf, sem)

    def _scatter(j, buf):
        pltpu.sync_copy(buf, agg_sh.at[col_v.at[j]], add=True)

    _gather(jnp.int32(0), buf_a, sem_a).wait()

    @pl.loop(jnp.int32(0), jnp.int32(NCHUNK - 1), step=jnp.int32(2))
    def _mainb(j):
        g_b = _gather(j + jnp.int32(1), buf_b, sem_b)
        _scatter(j, buf_a)
        g_b.wait()
        g_a = _gather(j + jnp.int32(2), buf_a, sem_a)
        _scatter(j + jnp.int32(1), buf_b)
        g_a.wait()

    _scatter(jnp.int32(NCHUNK - 1), buf_a)

    plsc.subcore_barrier()

    pltpu.sync_copy(agg_sh.at[pl.ds(sid * jnp.int32(RPT), RPT)],
                    out_hbm.at[cid, pl.ds(sid * jnp.int32(RPT), RPT)])


def _tail_body(raw_ref, xa_ref, s0_ref, s1_ref, d0_ref, d1_ref,
               wo_ref, bo_ref, wr_ref,
               g_ref, b_ref, wp_ref, bp_ref, wl_ref, bl_ref,
               out_ref, emb_ref):
    xb = xa_ref[...]
    s = s0_ref[0] + s1_ref[0]
    deg = d0_ref[...] + d1_ref[...]
    deg_inv = 1.0 / (deg + 1.0)
    agg = deg_inv * (s + xb)
    ne = (lax.dot_general(agg, wo_ref[...], (((1,), (1,)), ((), ())),
                          preferred_element_type=jnp.float32)
          + bo_ref[...]
          + lax.dot_general(xb, wr_ref[...], (((1,), (1,)), ((), ())),
                            preferred_element_type=jnp.float32))
    emb_ref[...] = ne
    h = _gelu_exact(ne)
    hc = jnp.concatenate([raw_ref[...], h], axis=1)
    mu = jnp.mean(hc, axis=-1, keepdims=True)
    var = jnp.mean((hc - mu) ** 2, axis=-1, keepdims=True)
    ln = (hc - mu) / jnp.sqrt(var + LN_EPS) * g_ref[...] + b_ref[...]
    h2 = _gelu_exact(
        lax.dot_general(ln, wp_ref[...], (((1,), (1,)), ((), ())),
                        preferred_element_type=jnp.float32) + bp_ref[...])
    out_ref[...] = jax.nn.sigmoid(
        lax.dot_general(h2, wl_ref[...], (((1,), (1,)), ((), ())),
                        preferred_element_type=jnp.float32) + bl_ref[...])


_full = lambda shape: pl.BlockSpec(shape, lambda i: (i * 0,) * len(shape))
_rows = lambda w: pl.BlockSpec((BROW, w), lambda i: (i, i * 0))

_mlp_call = pl.pallas_call(
    _mlp_body,
    grid=(N_NODES // BROW,),
    in_specs=[_rows(D_FEAT), _full((HIDDEN, D_FEAT)), _full((1, HIDDEN))],
    out_specs=_rows(D_FEAT),
    out_shape=jax.ShapeDtypeStruct((N_NODES, D_FEAT), jnp.float32),
)


@functools.cache
def _get_sc_calls():
    mesh = plsc.VectorSubcoreMesh(core_axis_name="c", subcore_axis_name="s",
                                  num_cores=NCORE)
    deg_call = functools.partial(
        pl.kernel,
        mesh=mesh,
        compiler_params=pltpu.CompilerParams(needs_layout_passes=False),
        out_type=jax.ShapeDtypeStruct((NCORE * AGG_ROWS,), jnp.float32),
        scratch_types=[
            pltpu.VMEM((NCHUNK, K), jnp.int32),
            pltpu.VMEM((NCHUNK, K), jnp.int32),
            pltpu.VMEM((AGG_ROWS,), jnp.float32),
            pltpu.VMEM((NSUB, CPT), jnp.float32),
            pltpu.VMEM((CPT,), jnp.float32),
            pltpu.VMEM_SHARED((NSUB, AGG_ROWS), jnp.float32),
        ],
    )(_deg_body)
    agg_call = functools.partial(
        pl.kernel,
        mesh=mesh,
        compiler_params=pltpu.CompilerParams(needs_layout_passes=False),
        out_type=jax.ShapeDtypeStruct((NCORE, AGG_ROWS, D_FEAT), jnp.float32),
        scratch_types=[
            pltpu.VMEM((EPT,), jnp.int32),
            pltpu.VMEM((NCHUNK, K), jnp.int32),
            pltpu.VMEM((K, D_FEAT), jnp.float32),
            pltpu.VMEM((K, D_FEAT), jnp.float32),
            pltpu.VMEM_SHARED((AGG_ROWS, D_FEAT), jnp.float32),
            pltpu.SemaphoreType.DMA,
            pltpu.SemaphoreType.DMA,
        ],
    )(_agg_body)
    return deg_call, agg_call


_tail_call = pl.pallas_call(
    _tail_body,
    grid=(N_NODES // BROW,),
    in_specs=[
        _rows(D_FEAT),
        _rows(D_FEAT),
        pl.BlockSpec((1, BROW, D_FEAT), lambda i: (i * 0, i, i * 0)),
        pl.BlockSpec((1, BROW, D_FEAT), lambda i: (i * 0 + 1, i, i * 0)),
        _rows(1),
        _rows(1),
        _full((HIDDEN, HIDDEN)),
        _full((1, HIDDEN)),
        _full((HIDDEN, HIDDEN)),
        _full((1, D_FEAT + HIDDEN)),
        _full((1, D_FEAT + HIDDEN)),
        _full((HIDDEN, D_FEAT + HIDDEN)),
        _full((1, HIDDEN)),
        _full((N_CLASSES, HIDDEN)),
        _full((1, N_CLASSES)),
    ],
    out_specs=[_rows(N_CLASSES), _rows(HIDDEN)],
    out_shape=[
        jax.ShapeDtypeStruct((N_NODES, N_CLASSES), jnp.float32),
        jax.ShapeDtypeStruct((N_NODES, HIDDEN), jnp.float32),
    ],
)


def kernel(raw, edge_index, W_mlp, b_mlp, W_out, b_out, W_root, ln_g, ln_b,
           W_post, b_post, W_lin1, b_lin1):
    ei = edge_index.astype(jnp.int32)
    row2 = ei[0].reshape(NW, EPT)
    row3 = ei[0].reshape(NW, NCHUNK, K)
    col3 = ei[1].reshape(NW, NCHUNK, K)

    x = _mlp_call(raw, W_mlp, b_mlp.reshape(1, HIDDEN))

    deg_call, agg_call = _get_sc_calls()
    deg_flat = deg_call(row3, col3)
    s3 = agg_call(x, row2, col3)
    degr = deg_flat.reshape(NCORE, AGG_ROWS)
    d0 = degr[0, :N_NODES].reshape(N_NODES, 1)
    d1 = degr[1, :N_NODES].reshape(N_NODES, 1)

    out, node_emb = _tail_call(
        raw, x, s3, s3, d0, d1,
        W_out, b_out.reshape(1, HIDDEN), W_root,
        ln_g.reshape(1, D_FEAT + HIDDEN), ln_b.reshape(1, D_FEAT + HIDDEN),
        W_post, b_post.reshape(1, HIDDEN),
        W_lin1, b_lin1.reshape(1, N_CLASSES))
    return out, node_emb


# speedup vs baseline: 19.9357x; 1.0121x over previous
import functools

import jax
import jax.numpy as jnp
from jax import lax
from jax.experimental import pallas as pl
from jax.experimental.pallas import tpu as pltpu
from jax.experimental.pallas import tpu_sc as plsc

N_NODES = 10000
N_EDGES = 320000
D_FEAT = 128
HIDDEN = 128
N_CLASSES = 64
LN_EPS = 1e-5

NCORE = 2
NSUB = 16
NW = NCORE * NSUB
EPT = N_EDGES // NW
K = 80
NCHUNK = EPT // K
KD = 80
NCHUNKD = EPT // KD
AGG_ROWS = 10240
RPT = AGG_ROWS // NSUB
CPT = AGG_ROWS // NSUB

BROW = 1000

_INV_SQRT2 = 0.7071067811865476


def _gelu_exact(x):
    return 0.5 * x * (1.0 + lax.erf(x * _INV_SQRT2))


def _mlp_body(raw_ref, wm_ref, bm_ref, out_ref):
    xb = lax.dot_general(raw_ref[...], wm_ref[...],
                         (((1,), (1,)), ((), ())),
                         preferred_element_type=jnp.float32)
    out_ref[...] = _gelu_exact(xb + bm_ref[...])


EROWS = N_EDGES // D_FEAT


def _colmod_body(row_ref, col_ref, cm_ref):
    r = row_ref[...]
    c = col_ref[...]
    cm_ref[...] = jnp.where(r == c, jnp.int32(N_NODES), c)


_colmod_call = pl.pallas_call(
    _colmod_body,
    in_specs=[pl.BlockSpec((EROWS, D_FEAT), lambda: (0, 0)),
              pl.BlockSpec((EROWS, D_FEAT), lambda: (0, 0))],
    out_specs=pl.BlockSpec((EROWS, D_FEAT), lambda: (0, 0)),
    out_shape=jax.ShapeDtypeStruct((EROWS, D_FEAT), jnp.int32),
)


def _deg_body(cm_hbm, deg_hbm, col_v, deg_v, dtmp_v, dres_v,
              stage_sh):
    cid = lax.axis_index("c")
    sid = lax.axis_index("s")
    wid = cid * jnp.int32(NSUB) + sid

    pltpu.sync_copy(cm_hbm.at[wid], col_v)

    zero = jnp.zeros((16,), jnp.float32)
    ones = jnp.ones((16,), jnp.float32)

    @pl.loop(jnp.int32(0), jnp.int32(AGG_ROWS // 16))
    def _zd(i):
        deg_v[pl.ds(i * jnp.int32(16), 16)] = zero

    @pl.loop(jnp.int32(0), jnp.int32(NCHUNKD))
    def _hist(j):
        for k2 in range(KD // 16):
            sl = pl.ds(jnp.int32(k2 * 16), 16)
            cc = col_v[j, sl]
            plsc.addupdate_scatter(deg_v, [cc], ones)

    pltpu.sync_copy(deg_v, stage_sh.at[sid])
    plsc.subcore_barrier()

    cbase = sid * jnp.int32(CPT)
    for t in range(NSUB):
        pltpu.sync_copy(stage_sh.at[jnp.int32(t), pl.ds(cbase, CPT)],
                        dtmp_v.at[jnp.int32(t)])

    @pl.loop(jnp.int32(0), jnp.int32(CPT // 16))
    def _dsum(k2):
        base = k2 * jnp.int32(16)
        acc = dtmp_v[jnp.int32(0), pl.ds(base, 16)]
        for t in range(1, NSUB):
            acc = acc + dtmp_v[jnp.int32(t), pl.ds(base, 16)]
        dres_v[pl.ds(base, 16)] = acc

    pltpu.sync_copy(dres_v,
                    deg_hbm.at[pl.ds(cid * jnp.int32(AGG_ROWS) + cbase, CPT)])


def _agg_body(x_hbm, row_hbm, col_hbm, out_hbm,
              row_v, col_v, buf_a, buf_b, agg_sh, sem_a, sem_b):
    cid = lax.axis_index("c")
    sid = lax.axis_index("s")
    wid = cid * jnp.int32(NSUB) + sid

    pltpu.sync_copy(row_hbm.at[wid], row_v)
    pltpu.sync_copy(col_hbm.at[wid], col_v)

    zero = jnp.zeros((16,), jnp.float32)

    @pl.loop(jnp.int32(0), jnp.int32(K))
    def _zb(i):
        for k2 in range(D_FEAT // 16):
            buf_a[i, pl.ds(jnp.int32(k2 * 16), 16)] = zero

    for b in range(RPT // K):
        pltpu.sync_copy(
            buf_a, agg_sh.at[pl.ds(sid * jnp.int32(RPT) + jnp.int32(b * K), K)])

    plsc.subcore_barrier()

    def _gather(j, buf, sem):
        return pltpu.async_copy(x_hbm.at[row_v.at[pl.ds(j * jnp.int32(K), K)]],
                                buf, sem)

    def _scatter(j, buf):
        pltpu.sync_copy(buf, agg_sh.at[col_v.at[j]], add=True)

    _gather(jnp.int32(0), buf_a, sem_a).wait()

    @pl.loop(jnp.int32(0), jnp.int32(NCHUNK - 1), step=jnp.int32(2))
    def _mainb(j):
        g_b = _gather(j + jnp.int32(1), buf_b, sem_b)
        _scatter(j, buf_a)
        g_b.wait()
        g_a = _gather(j + jnp.int32(2), buf_a, sem_a)
        _scatter(j + jnp.int32(1), buf_b)
        g_a.wait()

    _scatter(jnp.int32(NCHUNK - 1), buf_a)

    plsc.subcore_barrier()

    pltpu.sync_copy(agg_sh.at[pl.ds(sid * jnp.int32(RPT), RPT)],
                    out_hbm.at[cid, pl.ds(sid * jnp.int32(RPT), RPT)])


def _tail_body(raw_ref, xa_ref, s0_ref, s1_ref, d0_ref, d1_ref,
               wo_ref, bo_ref, wr_ref,
               g_ref, b_ref, wp_ref, bp_ref, wl_ref, bl_ref,
               out_ref, emb_ref):
    xb = xa_ref[...]
    s = s0_ref[0] + s1_ref[0]
    deg = d0_ref[...] + d1_ref[...]
    deg_inv = 1.0 / (deg + 1.0)
    agg = deg_inv * (s + xb)
    ne = (lax.dot_general(agg, wo_ref[...], (((1,), (1,)), ((), ())),
                          preferred_element_type=jnp.float32)
          + bo_ref[...]
          + lax.dot_general(xb, wr_ref[...], (((1,), (1,)), ((), ())),
                            preferred_element_type=jnp.float32))
    emb_ref[...] = ne
    h = _gelu_exact(ne)
    hc = jnp.concatenate([raw_ref[...], h], axis=1)
    mu = jnp.mean(hc, axis=-1, keepdims=True)
    var = jnp.mean((hc - mu) ** 2, axis=-1, keepdims=True)
    ln = (hc - mu) / jnp.sqrt(var + LN_EPS) * g_ref[...] + b_ref[...]
    h2 = _gelu_exact(
        lax.dot_general(ln, wp_ref[...], (((1,), (1,)), ((), ())),
                        preferred_element_type=jnp.float32) + bp_ref[...])
    out_ref[...] = jax.nn.sigmoid(
        lax.dot_general(h2, wl_ref[...], (((1,), (1,)), ((), ())),
                        preferred_element_type=jnp.float32) + bl_ref[...])


_full = lambda shape: pl.BlockSpec(shape, lambda i: (i * 0,) * len(shape))
_rows = lambda w: pl.BlockSpec((BROW, w), lambda i: (i, i * 0))

_mlp_call = pl.pallas_call(
    _mlp_body,
    grid=(N_NODES // BROW,),
    in_specs=[_rows(D_FEAT), _full((HIDDEN, D_FEAT)), _full((1, HIDDEN))],
    out_specs=_rows(D_FEAT),
    out_shape=jax.ShapeDtypeStruct((N_NODES, D_FEAT), jnp.float32),
)


@functools.cache
def _get_sc_calls():
    mesh = plsc.VectorSubcoreMesh(core_axis_name="c", subcore_axis_name="s",
                                  num_cores=NCORE)
    deg_call = functools.partial(
        pl.kernel,
        mesh=mesh,
        compiler_params=pltpu.CompilerParams(needs_layout_passes=False),
        out_type=jax.ShapeDtypeStruct((NCORE * AGG_ROWS,), jnp.float32),
        scratch_types=[
            pltpu.VMEM((NCHUNKD, KD), jnp.int32),
            pltpu.VMEM((AGG_ROWS,), jnp.float32),
            pltpu.VMEM((NSUB, CPT), jnp.float32),
            pltpu.VMEM((CPT,), jnp.float32),
            pltpu.VMEM_SHARED((NSUB, AGG_ROWS), jnp.float32),
        ],
    )(_deg_body)
    agg_call = functools.partial(
        pl.kernel,
        mesh=mesh,
        compiler_params=pltpu.CompilerParams(needs_layout_passes=False),
        out_type=jax.ShapeDtypeStruct((NCORE, AGG_ROWS, D_FEAT), jnp.float32),
        scratch_types=[
            pltpu.VMEM((EPT,), jnp.int32),
            pltpu.VMEM((NCHUNK, K), jnp.int32),
            pltpu.VMEM((K, D_FEAT), jnp.float32),
            pltpu.VMEM((K, D_FEAT), jnp.float32),
            pltpu.VMEM_SHARED((AGG_ROWS, D_FEAT), jnp.float32),
            pltpu.SemaphoreType.DMA,
            pltpu.SemaphoreType.DMA,
        ],
    )(_agg_body)
    return deg_call, agg_call


_tail_call = pl.pallas_call(
    _tail_body,
    grid=(N_NODES // BROW,),
    in_specs=[
        _rows(D_FEAT),
        _rows(D_FEAT),
        pl.BlockSpec((1, BROW, D_FEAT), lambda i: (i * 0, i, i * 0)),
        pl.BlockSpec((1, BROW, D_FEAT), lambda i: (i * 0 + 1, i, i * 0)),
        _rows(1),
        _rows(1),
        _full((HIDDEN, HIDDEN)),
        _full((1, HIDDEN)),
        _full((HIDDEN, HIDDEN)),
        _full((1, D_FEAT + HIDDEN)),
        _full((1, D_FEAT + HIDDEN)),
        _full((HIDDEN, D_FEAT + HIDDEN)),
        _full((1, HIDDEN)),
        _full((N_CLASSES, HIDDEN)),
        _full((1, N_CLASSES)),
    ],
    out_specs=[_rows(N_CLASSES), _rows(HIDDEN)],
    out_shape=[
        jax.ShapeDtypeStruct((N_NODES, N_CLASSES), jnp.float32),
        jax.ShapeDtypeStruct((N_NODES, HIDDEN), jnp.float32),
    ],
)


def kernel(raw, edge_index, W_mlp, b_mlp, W_out, b_out, W_root, ln_g, ln_b,
           W_post, b_post, W_lin1, b_lin1):
    ei = edge_index.astype(jnp.int32)
    row2 = ei[0].reshape(NW, EPT)
    cm = _colmod_call(ei[0].reshape(EROWS, D_FEAT),
                      ei[1].reshape(EROWS, D_FEAT))
    cm_d = cm.reshape(NW, NCHUNKD, KD)
    cm_a = cm.reshape(NW, NCHUNK, K)

    x = _mlp_call(raw, W_mlp, b_mlp.reshape(1, HIDDEN))

    deg_call, agg_call = _get_sc_calls()
    deg_flat = deg_call(cm_d)
    s3 = agg_call(x, row2, cm_a)
    degr = deg_flat.reshape(NCORE, AGG_ROWS)
    d0 = degr[0, :N_NODES].reshape(N_NODES, 1)
    d1 = degr[1, :N_NODES].reshape(N_NODES, 1)

    out, node_emb = _tail_call(
        raw, x, s3, s3, d0, d1,
        W_out, b_out.reshape(1, HIDDEN), W_root,
        ln_g.reshape(1, D_FEAT + HIDDEN), ln_b.reshape(1, D_FEAT + HIDDEN),
        W_post, b_post.reshape(1, HIDDEN),
        W_lin1, b_lin1.reshape(1, N_CLASSES))
    return out, node_emb


# speedup vs baseline: 20.2302x; 1.0148x over previous
import functools

import jax
import jax.numpy as jnp
from jax import lax
from jax.experimental import pallas as pl
from jax.experimental.pallas import tpu as pltpu
from jax.experimental.pallas import tpu_sc as plsc

N_NODES = 10000
N_EDGES = 320000
D_FEAT = 128
HIDDEN = 128
N_CLASSES = 64
LN_EPS = 1e-5

NCORE = 2
NSUB = 16
NW = NCORE * NSUB
EPT = N_EDGES // NW
K = 80
NCHUNK = EPT // K
KD = 80
NCHUNKD = EPT // KD
AGG_ROWS = 10240
RPT = AGG_ROWS // NSUB
CPT = AGG_ROWS // NSUB

BROW = 2000

_INV_SQRT2 = 0.7071067811865476


def _gelu_exact(x):
    return 0.5 * x * (1.0 + lax.erf(x * _INV_SQRT2))


def _mlp_body(raw_ref, wm_ref, bm_ref, out_ref):
    xb = lax.dot_general(raw_ref[...], wm_ref[...],
                         (((1,), (1,)), ((), ())),
                         preferred_element_type=jnp.float32)
    out_ref[...] = _gelu_exact(xb + bm_ref[...])


EROWS = N_EDGES // D_FEAT


def _colmod_body(row_ref, col_ref, cm_ref):
    r = row_ref[...]
    c = col_ref[...]
    cm_ref[...] = jnp.where(r == c, jnp.int32(N_NODES), c)


_colmod_call = pl.pallas_call(
    _colmod_body,
    in_specs=[pl.BlockSpec((EROWS, D_FEAT), lambda: (0, 0)),
              pl.BlockSpec((EROWS, D_FEAT), lambda: (0, 0))],
    out_specs=pl.BlockSpec((EROWS, D_FEAT), lambda: (0, 0)),
    out_shape=jax.ShapeDtypeStruct((EROWS, D_FEAT), jnp.int32),
)


def _deg_body(cm_hbm, deg_hbm, col_v, deg_v, dtmp_v, dres_v,
              stage_sh):
    cid = lax.axis_index("c")
    sid = lax.axis_index("s")
    wid = cid * jnp.int32(NSUB) + sid

    pltpu.sync_copy(cm_hbm.at[wid], col_v)

    zero = jnp.zeros((16,), jnp.float32)
    ones = jnp.ones((16,), jnp.float32)

    @pl.loop(jnp.int32(0), jnp.int32(AGG_ROWS // 16))
    def _zd(i):
        deg_v[pl.ds(i * jnp.int32(16), 16)] = zero

    @pl.loop(jnp.int32(0), jnp.int32(NCHUNKD))
    def _hist(j):
        for k2 in range(KD // 16):
            sl = pl.ds(jnp.int32(k2 * 16), 16)
            cc = col_v[j, sl]
            plsc.addupdate_scatter(deg_v, [cc], ones)

    pltpu.sync_copy(deg_v, stage_sh.at[sid])
    plsc.subcore_barrier()

    cbase = sid * jnp.int32(CPT)
    for t in range(NSUB):
        pltpu.sync_copy(stage_sh.at[jnp.int32(t), pl.ds(cbase, CPT)],
                        dtmp_v.at[jnp.int32(t)])

    @pl.loop(jnp.int32(0), jnp.int32(CPT // 16))
    def _dsum(k2):
        base = k2 * jnp.int32(16)
        acc = dtmp_v[jnp.int32(0), pl.ds(base, 16)]
        for t in range(1, NSUB):
            acc = acc + dtmp_v[jnp.int32(t), pl.ds(base, 16)]
        dres_v[pl.ds(base, 16)] = acc

    pltpu.sync_copy(dres_v,
                    deg_hbm.at[pl.ds(cid * jnp.int32(AGG_ROWS) + cbase, CPT)])


def _agg_body(x_hbm, row_hbm, col_hbm, out_hbm,
              row_v, col_v, buf_a, buf_b, agg_sh, sem_a, sem_b):
    cid = lax.axis_index("c")
    sid = lax.axis_index("s")
    wid = cid * jnp.int32(NSUB) + sid

    pltpu.sync_copy(row_hbm.at[wid], row_v)
    pltpu.sync_copy(col_hbm.at[wid], col_v)

    zero = jnp.zeros((16,), jnp.float32)

    @pl.loop(jnp.int32(0), jnp.int32(K))
    def _zb(i):
        for k2 in range(D_FEAT // 16):
            buf_a[i, pl.ds(jnp.int32(k2 * 16), 16)] = zero

    for b in range(RPT // K):
        pltpu.sync_copy(
            buf_a, agg_sh.at[pl.ds(sid * jnp.int32(RPT) + jnp.int32(b * K), K)])

    plsc.subcore_barrier()

    def _gather(j, buf, sem):
        return pltpu.async_copy(x_hbm.at[row_v.at[pl.ds(j * jnp.int32(K), K)]],
                                buf, sem)

    def _scatter(j, buf):
        pltpu.sync_copy(buf, agg_sh.at[col_v.at[j]], add=True)

    _gather(jnp.int32(0), buf_a, sem_a).wait()

    @pl.loop(jnp.int32(0), jnp.int32(NCHUNK - 1), step=jnp.int32(2))
    def _mainb(j):
        g_b = _gather(j + jnp.int32(1), buf_b, sem_b)
        _scatter(j, buf_a)
        g_b.wait()
        g_a = _gather(j + jnp.int32(2), buf_a, sem_a)
        _scatter(j + jnp.int32(1), buf_b)
        g_a.wait()

    _scatter(jnp.int32(NCHUNK - 1), buf_a)

    plsc.subcore_barrier()

    pltpu.sync_copy(agg_sh.at[pl.ds(sid * jnp.int32(RPT), RPT)],
                    out_hbm.at[cid, pl.ds(sid * jnp.int32(RPT), RPT)])


def _tail_body(raw_ref, xa_ref, s0_ref, s1_ref, d0_ref, d1_ref,
               wo_ref, bo_ref, wr_ref,
               g_ref, b_ref, wp_ref, bp_ref, wl_ref, bl_ref,
               out_ref, emb_ref):
    xb = xa_ref[...]
    s = s0_ref[0] + s1_ref[0]
    deg = d0_ref[...] + d1_ref[...]
    deg_inv = 1.0 / (deg + 1.0)
    agg = deg_inv * (s + xb)
    ne = (lax.dot_general(agg, wo_ref[...], (((1,), (1,)), ((), ())),
                          preferred_element_type=jnp.float32)
          + bo_ref[...]
          + lax.dot_general(xb, wr_ref[...], (((1,), (1,)), ((), ())),
                            preferred_element_type=jnp.float32))
    emb_ref[...] = ne
    h = _gelu_exact(ne)
    hc = jnp.concatenate([raw_ref[...], h], axis=1)
    mu = jnp.mean(hc, axis=-1, keepdims=True)
    var = jnp.mean((hc - mu) ** 2, axis=-1, keepdims=True)
    ln = (hc - mu) / jnp.sqrt(var + LN_EPS) * g_ref[...] + b_ref[...]
    h2 = _gelu_exact(
        lax.dot_general(ln, wp_ref[...], (((1,), (1,)), ((), ())),
                        preferred_element_type=jnp.float32) + bp_ref[...])
    out_ref[...] = jax.nn.sigmoid(
        lax.dot_general(h2, wl_ref[...], (((1,), (1,)), ((), ())),
                        preferred_element_type=jnp.float32) + bl_ref[...])


_full = lambda shape: pl.BlockSpec(shape, lambda i: (i * 0,) * len(shape))
_rows = lambda w: pl.BlockSpec((BROW, w), lambda i: (i, i * 0))

_mlp_call = pl.pallas_call(
    _mlp_body,
    grid=(N_NODES // BROW,),
    in_specs=[_rows(D_FEAT), _full((HIDDEN, D_FEAT)), _full((1, HIDDEN))],
    out_specs=_rows(D_FEAT),
    out_shape=jax.ShapeDtypeStruct((N_NODES, D_FEAT), jnp.float32),
)


@functools.cache
def _get_sc_calls():
    mesh = plsc.VectorSubcoreMesh(core_axis_name="c", subcore_axis_name="s",
                                  num_cores=NCORE)
    deg_call = functools.partial(
        pl.kernel,
        mesh=mesh,
        compiler_params=pltpu.CompilerParams(needs_layout_passes=False),
        out_type=jax.ShapeDtypeStruct((NCORE * AGG_ROWS,), jnp.float32),
        scratch_types=[
            pltpu.VMEM((NCHUNKD, KD), jnp.int32),
            pltpu.VMEM((AGG_ROWS,), jnp.float32),
            pltpu.VMEM((NSUB, CPT), jnp.float32),
            pltpu.VMEM((CPT,), jnp.float32),
            pltpu.VMEM_SHARED((NSUB, AGG_ROWS), jnp.float32),
        ],
    )(_deg_body)
    agg_call = functools.partial(
        pl.kernel,
        mesh=mesh,
        compiler_params=pltpu.CompilerParams(needs_layout_passes=False),
        out_type=jax.ShapeDtypeStruct((NCORE, AGG_ROWS, D_FEAT), jnp.float32),
        scratch_types=[
            pltpu.VMEM((EPT,), jnp.int32),
            pltpu.VMEM((NCHUNK, K), jnp.int32),
            pltpu.VMEM((K, D_FEAT), jnp.float32),
            pltpu.VMEM((K, D_FEAT), jnp.float32),
            pltpu.VMEM_SHARED((AGG_ROWS, D_FEAT), jnp.float32),
            pltpu.SemaphoreType.DMA,
            pltpu.SemaphoreType.DMA,
        ],
    )(_agg_body)
    return deg_call, agg_call


_tail_call = pl.pallas_call(
    _tail_body,
    grid=(N_NODES // BROW,),
    in_specs=[
        _rows(D_FEAT),
        _rows(D_FEAT),
        pl.BlockSpec((1, BROW, D_FEAT), lambda i: (i * 0, i, i * 0)),
        pl.BlockSpec((1, BROW, D_FEAT), lambda i: (i * 0 + 1, i, i * 0)),
        _rows(1),
        _rows(1),
        _full((HIDDEN, HIDDEN)),
        _full((1, HIDDEN)),
        _full((HIDDEN, HIDDEN)),
        _full((1, D_FEAT + HIDDEN)),
        _full((1, D_FEAT + HIDDEN)),
        _full((HIDDEN, D_FEAT + HIDDEN)),
        _full((1, HIDDEN)),
        _full((N_CLASSES, HIDDEN)),
        _full((1, N_CLASSES)),
    ],
    out_specs=[_rows(N_CLASSES), _rows(HIDDEN)],
    out_shape=[
        jax.ShapeDtypeStruct((N_NODES, N_CLASSES), jnp.float32),
        jax.ShapeDtypeStruct((N_NODES, HIDDEN), jnp.float32),
    ],
)


def kernel(raw, edge_index, W_mlp, b_mlp, W_out, b_out, W_root, ln_g, ln_b,
           W_post, b_post, W_lin1, b_lin1):
    ei = edge_index.astype(jnp.int32)
    row2 = ei[0].reshape(NW, EPT)
    cm = _colmod_call(ei[0].reshape(EROWS, D_FEAT),
                      ei[1].reshape(EROWS, D_FEAT))
    cm_d = cm.reshape(NW, NCHUNKD, KD)
    cm_a = cm.reshape(NW, NCHUNK, K)

    x = _mlp_call(raw, W_mlp, b_mlp.reshape(1, HIDDEN))

    deg_call, agg_call = _get_sc_calls()
    deg_flat = deg_call(cm_d)
    s3 = agg_call(x, row2, cm_a)
    degr = deg_flat.reshape(NCORE, AGG_ROWS)
    d0 = degr[0, :N_NODES].reshape(N_NODES, 1)
    d1 = degr[1, :N_NODES].reshape(N_NODES, 1)

    out, node_emb = _tail_call(
        raw, x, s3, s3, d0, d1,
        W_out, b_out.reshape(1, HIDDEN), W_root,
        ln_g.reshape(1, D_FEAT + HIDDEN), ln_b.reshape(1, D_FEAT + HIDDEN),
        W_post, b_post.reshape(1, HIDDEN),
        W_lin1, b_lin1.reshape(1, N_CLASSES))
    return out, node_emb
